# baseline (speedup 1.0000x reference)
import functools

import jax
import jax.numpy as jnp
from jax import lax
from jax.experimental import pallas as pl
from jax.experimental.pallas import tpu as pltpu
from jax.experimental.pallas import tpu_sc as plsc

NUM_CORES = 2
NUM_SUBCORES = 16
NUM_WORKERS = NUM_CORES * NUM_SUBCORES
LANES = 16


def kernel(input_ids, table):
    seq, batch = input_ids.shape
    vocab, dim = table.shape
    assert batch == NUM_WORKERS * 128
    assert 2 * dim == 128
    assert vocab % 128 in (0, 64) and vocab % 2 == 0
    full_blocks = vocab // 128
    nblk = full_blocks + (1 if vocab % 128 else 0)
    n_iter = (nblk + NUM_WORKERS - 1) // NUM_WORKERS

    ids = input_ids.astype(jnp.int32)
    tableT = table.T
    tailT = table[vocab - 128:, :].T

    mesh = plsc.VectorSubcoreMesh(
        core_axis_name="c", subcore_axis_name="s",
        num_cores=NUM_CORES, num_subcores=NUM_SUBCORES)

    @functools.partial(
        pl.kernel,
        mesh=mesh,
        out_type=jax.ShapeDtypeStruct((vocab // 2, 128), jnp.float32),
        scratch_types=[
            pltpu.VMEM((dim, 128), jnp.float32),
            pltpu.VMEM((dim, 128), jnp.float32),
        ],
        compiler_params=pltpu.CompilerParams(needs_layout_passes=False),
    )
    def transpose_k(tt_hbm, tail_hbm, r_hbm, tbuf, obuf):
        wid = lax.axis_index("s") * NUM_CORES + lax.axis_index("c")
        rows16 = [lax.iota(jnp.int32, LANES) + LANES * q for q in range(4)]

        def transpose_block(rbase):
            def pbody(p, c2):
                for g in range(8):
                    cvec = jnp.zeros((LANES,), jnp.int32) + (2 * p + g // 4)
                    vec = plsc.load_gather(tbuf, [rows16[g % 4], cvec])
                    obuf[p, pl.ds(g * LANES, LANES)] = vec
                return c2

            lax.fori_loop(0, 64, pbody, 0)
            pltpu.sync_copy(obuf, r_hbm.at[pl.ds(rbase, 64), :])

        def item(k, carry):
            v = k * NUM_WORKERS + wid

            @pl.when(v < full_blocks)
            def _():
                src = pl.multiple_of(v * 128, 128)
                pltpu.sync_copy(tt_hbm.at[:, pl.ds(src, 128)], tbuf)
                transpose_block(pl.multiple_of(v * 64, 8))

            if vocab % 128:
                @pl.when(v == full_blocks)
                def _():
                    pltpu.sync_copy(tail_hbm, tbuf)
                    transpose_block((vocab - 128) // 2)

            return carry

        lax.fori_loop(0, n_iter, item, 0)

    @functools.partial(
        pl.kernel,
        mesh=mesh,
        out_type=jax.ShapeDtypeStruct((seq, dim, batch), jnp.float32),
        scratch_types=[
            pltpu.VMEM((seq, 128), jnp.int32),
            pltpu.VMEM((seq, 128), jnp.int32),
            pltpu.VMEM((seq, 128), jnp.int32),
            pltpu.VMEM((128, 128), jnp.float32),
            pltpu.VMEM((dim, 128), jnp.float32),
            pltpu.SemaphoreType.DMA,
        ],
        compiler_params=pltpu.CompilerParams(needs_layout_passes=False),
    )
    def gather_k(ids_hbm, r_hbm, out_hbm, idxv, idx2, hoff, gbuf, obuf, gsem):
        wid = lax.axis_index("s") * NUM_CORES + lax.axis_index("c")
        col = wid * 128
        rows16 = [lax.iota(jnp.int32, LANES) + LANES * g for g in range(8)]
        pltpu.sync_copy(ids_hbm.at[:, pl.ds(col, 128)], idxv)

        def prep(s, carry):
            for g in range(8):
                v = idxv[s, pl.ds(g * LANES, LANES)]
                idx2[s, pl.ds(g * LANES, LANES)] = v >> 1
                hoff[s, pl.ds(g * LANES, LANES)] = (v & 1) * dim
            return carry

        lax.fori_loop(0, seq, prep, 0)

        def step(s, carry):
            pltpu.async_copy(r_hbm.at[idx2.at[s]], gbuf, gsem).wait()

            def crow(c, c2):
                for g in range(8):
                    hv = hoff[s, pl.ds(g * LANES, LANES)]
                    vec = plsc.load_gather(gbuf, [rows16[g], hv + c])
                    obuf[c, pl.ds(g * LANES, LANES)] = vec
                return c2

            lax.fori_loop(0, dim, crow, 0)
            pltpu.sync_copy(obuf, out_hbm.at[s].at[:, pl.ds(col, 128)])
            return carry

        lax.fori_loop(0, seq, step, 0)

    r_pack = transpose_k(tableT, tailT)
    out_phys = gather_k(ids, r_pack)
    return jnp.transpose(out_phys, (0, 2, 1))


# speedup vs baseline: 2.3413x; 2.3413x over previous
import functools

import jax
import jax.numpy as jnp
from jax import lax
from jax.experimental import pallas as pl
from jax.experimental.pallas import tpu as pltpu
from jax.experimental.pallas import tpu_sc as plsc

NUM_CORES = 2
NUM_SUBCORES = 16
NUM_WORKERS = NUM_CORES * NUM_SUBCORES
LANES = 16


def kernel(input_ids, table):
    seq, batch = input_ids.shape
    vocab, dim = table.shape
    assert batch == NUM_WORKERS * 128
    assert 2 * dim == 128

    ids = input_ids.astype(jnp.int32)
    rdup = jnp.pad(table, ((0, 0), (0, 128 - dim)))

    mesh = plsc.VectorSubcoreMesh(
        core_axis_name="c", subcore_axis_name="s",
        num_cores=NUM_CORES, num_subcores=NUM_SUBCORES)

    @functools.partial(
        pl.kernel,
        mesh=mesh,
        out_type=jax.ShapeDtypeStruct((seq, dim, batch), jnp.float32),
        scratch_types=[
            pltpu.VMEM((seq, 128), jnp.int32),
            pltpu.VMEM((128, 128), jnp.float32),
            pltpu.VMEM((128, 128), jnp.float32),
            pltpu.VMEM((dim, 128), jnp.float32),
            pltpu.VMEM((dim, 128), jnp.float32),
            pltpu.SemaphoreType.DMA,
            pltpu.SemaphoreType.DMA,
            pltpu.SemaphoreType.DMA,
            pltpu.SemaphoreType.DMA,
        ],
        compiler_params=pltpu.CompilerParams(needs_layout_passes=False),
    )
    def gather_k(ids_hbm, r_hbm, out_hbm, idxv, gb0, gb1, ob0, ob1,
                 gs0, gs1, ss0, ss1):
        gbufs, obufs = (gb0, gb1), (ob0, ob1)
        gsems, ssems = (gs0, gs1), (ss0, ss1)
        wid = lax.axis_index("s") * NUM_CORES + lax.axis_index("c")
        col = wid * 128
        rows16 = [lax.iota(jnp.int32, LANES) + LANES * g for g in range(8)]
        pltpu.sync_copy(ids_hbm.at[:, pl.ds(col, 128)], idxv)

        def start_gather(s, b):
            pltpu.async_copy(r_hbm.at[idxv.at[s]], gbufs[b], gsems[b])

        def wait_gather(s, b):
            pltpu.make_async_copy(
                r_hbm.at[idxv.at[s]], gbufs[b], gsems[b]).wait()

        def dst(s):
            return out_hbm.at[s].at[:, pl.ds(col, 128)]

        def start_store(s, b):
            pltpu.async_copy(obufs[b], dst(s), ssems[b])

        def wait_store(s, b):
            pltpu.make_async_copy(obufs[b], dst(s), ssems[b]).wait()

        def transpose(b):
            gbuf, obuf = gbufs[b], obufs[b]

            def crow(c4, c2):
                for u in range(4):
                    c = c4 * 4 + u
                    cvec = jnp.zeros((LANES,), jnp.int32) + c
                    for g in range(8):
                        vec = plsc.load_gather(gbuf, [rows16[g], cvec])
                        obuf[c, pl.ds(g * LANES, LANES)] = vec
                return c2

            lax.fori_loop(0, dim // 4, crow, 0, unroll=2)

        start_gather(0, 0)

        def step(s, carry):
            b = lax.rem(s, 2)

            @pl.when(b == 0)
            def _():
                wait_gather(s, 0)

                @pl.when(s + 1 < seq)
                def _():
                    start_gather(s + 1, 1)

                @pl.when(s >= 2)
                def _():
                    wait_store(s - 2, 0)
                transpose(0)
                start_store(s, 0)

            @pl.when(b == 1)
            def _():
                wait_gather(s, 1)

                @pl.when(s + 1 < seq)
                def _():
                    start_gather(s + 1, 0)

                @pl.when(s >= 2)
                def _():
                    wait_store(s - 2, 1)
                transpose(1)
                start_store(s, 1)

            return carry

        lax.fori_loop(0, seq, step, 0)
        wait_store(seq - 2, (seq - 2) % 2)
        wait_store(seq - 1, (seq - 1) % 2)

    out_phys = gather_k(ids, rdup)
    return jnp.transpose(out_phys, (0, 2, 1))


# speedup vs baseline: 2.7032x; 1.1546x over previous
import functools

import jax
import jax.numpy as jnp
from jax import lax
from jax.experimental import pallas as pl
from jax.experimental.pallas import tpu as pltpu
from jax.experimental.pallas import tpu_sc as plsc

NUM_CORES = 2
NUM_SUBCORES = 16
NUM_WORKERS = NUM_CORES * NUM_SUBCORES
LANES = 16


def kernel(input_ids, table):
    seq, batch = input_ids.shape
    vocab, dim = table.shape
    assert batch == NUM_WORKERS * 128
    assert 2 * dim == 128

    ids = input_ids.astype(jnp.int32)
    rdup = jnp.pad(table, ((0, 0), (0, 128 - dim)))

    mesh = plsc.VectorSubcoreMesh(
        core_axis_name="c", subcore_axis_name="s",
        num_cores=NUM_CORES, num_subcores=NUM_SUBCORES)

    @functools.partial(
        pl.kernel,
        mesh=mesh,
        out_type=jax.ShapeDtypeStruct((seq, dim, batch), jnp.float32),
        scratch_types=[
            pltpu.VMEM((seq, 128), jnp.int32),
            pltpu.VMEM((128, 128), jnp.float32),
            pltpu.VMEM((128, 128), jnp.float32),
            pltpu.VMEM((dim, 128), jnp.float32),
            pltpu.VMEM((dim, 128), jnp.float32),
            pltpu.SemaphoreType.DMA,
            pltpu.SemaphoreType.DMA,
            pltpu.SemaphoreType.DMA,
            pltpu.SemaphoreType.DMA,
        ],
        compiler_params=pltpu.CompilerParams(needs_layout_passes=False),
    )
    def gather_k(ids_hbm, r_hbm, out_hbm, idxv, gb0, gb1, ob0, ob1,
                 gs0, gs1, ss0, ss1):
        gbufs, obufs = (gb0, gb1), (ob0, ob1)
        gsems, ssems = (gs0, gs1), (ss0, ss1)
        wid = lax.axis_index("s") * NUM_CORES + lax.axis_index("c")
        col = wid * 128
        rows16 = [lax.iota(jnp.int32, LANES) + LANES * g for g in range(8)]
        pltpu.sync_copy(ids_hbm.at[:, pl.ds(col, 128)], idxv)

        def start_gather(s, b):
            pltpu.async_copy(r_hbm.at[idxv.at[s]], gbufs[b], gsems[b])

        def wait_gather(s, b):
            pltpu.make_async_copy(
                r_hbm.at[idxv.at[s]], gbufs[b], gsems[b]).wait()

        def dst(s):
            return out_hbm.at[s].at[:, pl.ds(col, 128)]

        def start_store(s, b):
            pltpu.async_copy(obufs[b], dst(s), ssems[b])

        def wait_store(s, b):
            pltpu.make_async_copy(obufs[b], dst(s), ssems[b]).wait()

        def transpose(b):
            gbuf, obuf = gbufs[b], obufs[b]

            def jrow(j4, c2):
                for u in range(4):
                    j = j4 * 4 + u
                    jvec = jnp.zeros((LANES,), jnp.int32) + j
                    for q in range(dim // LANES):
                        vec = gbuf[j, pl.ds(q * LANES, LANES)]
                        plsc.store_scatter(obuf, [rows16[q], jvec], vec)
                return c2

            lax.fori_loop(0, 32, jrow, 0, unroll=2)

        start_gather(0, 0)

        def step(s, carry):
            b = lax.rem(s, 2)

            @pl.when(b == 0)
            def _():
                wait_gather(s, 0)

                @pl.when(s + 1 < seq)
                def _():
                    start_gather(s + 1, 1)

                @pl.when(s >= 2)
                def _():
                    wait_store(s - 2, 0)
                transpose(0)
                start_store(s, 0)

            @pl.when(b == 1)
            def _():
                wait_gather(s, 1)

                @pl.when(s + 1 < seq)
                def _():
                    start_gather(s + 1, 0)

                @pl.when(s >= 2)
                def _():
                    wait_store(s - 2, 1)
                transpose(1)
                start_store(s, 1)

            return carry

        lax.fori_loop(0, seq, step, 0)
        wait_store(seq - 2, (seq - 2) % 2)
        wait_store(seq - 1, (seq - 1) % 2)

    out_phys = gather_k(ids, rdup)
    return jnp.transpose(out_phys, (0, 2, 1))


# speedup vs baseline: 3.3167x; 1.2270x over previous
import functools

import jax
import jax.numpy as jnp
from jax import lax
from jax.experimental import pallas as pl
from jax.experimental.pallas import tpu as pltpu
from jax.experimental.pallas import tpu_sc as plsc

NUM_CORES = 2
NUM_SUBCORES = 16
NUM_WORKERS = NUM_CORES * NUM_SUBCORES
LANES = 16


def kernel(input_ids, table):
    seq, batch = input_ids.shape
    vocab, dim = table.shape
    assert batch == NUM_WORKERS * 128
    assert 2 * dim == 128

    ids = input_ids.astype(jnp.int32)
    rdup = jnp.pad(table, ((0, 0), (0, 128 - dim)))

    mesh = plsc.VectorSubcoreMesh(
        core_axis_name="c", subcore_axis_name="s",
        num_cores=NUM_CORES, num_subcores=NUM_SUBCORES)

    @functools.partial(
        pl.kernel,
        mesh=mesh,
        out_type=jax.ShapeDtypeStruct((seq, dim, batch), jnp.float32),
        scratch_types=[
            pltpu.VMEM((seq, 128), jnp.int32),
            pltpu.VMEM((128, 128), jnp.float32),
            pltpu.VMEM((128, 128), jnp.float32),
            pltpu.VMEM((dim, 128), jnp.float32),
            pltpu.VMEM((dim, 128), jnp.float32),
            pltpu.SemaphoreType.DMA,
            pltpu.SemaphoreType.DMA,
            pltpu.SemaphoreType.DMA,
            pltpu.SemaphoreType.DMA,
        ],
        compiler_params=pltpu.CompilerParams(needs_layout_passes=False),
    )
    def gather_k(ids_hbm, r_hbm, out_hbm, idxv, gb0, gb1, ob0, ob1,
                 gs0, gs1, ss0, ss1):
        gbufs, obufs = (gb0, gb1), (ob0, ob1)
        gsems, ssems = (gs0, gs1), (ss0, ss1)
        wid = lax.axis_index("s") * NUM_CORES + lax.axis_index("c")
        col = wid * 128
        rows16 = [lax.iota(jnp.int32, LANES) + LANES * g for g in range(8)]
        pltpu.sync_copy(ids_hbm.at[:, pl.ds(col, 128)], idxv)

        def start_gather(s, b):
            pltpu.async_copy(r_hbm.at[idxv.at[s]], gbufs[b], gsems[b])

        def wait_gather(s, b):
            pltpu.make_async_copy(
                r_hbm.at[idxv.at[s]], gbufs[b], gsems[b]).wait()

        def dst(s):
            return out_hbm.at[s].at[:, pl.ds(col, 128)]

        def start_store(s, b):
            pltpu.async_copy(obufs[b], dst(s), ssems[b])

        def wait_store(s, b):
            pltpu.make_async_copy(obufs[b], dst(s), ssems[b]).wait()

        def transpose(b):
            gbuf, obuf = gbufs[b], obufs[b]

            @plsc.parallel_loop(0, 32, unroll=4)
            def jrow(j4):
                for u in range(4):
                    j = j4 * 4 + u
                    jvec = jnp.zeros((LANES,), jnp.int32) + j
                    for q in range(dim // LANES):
                        vec = gbuf[j, pl.ds(q * LANES, LANES)]
                        plsc.store_scatter(obuf, [rows16[q], jvec], vec)

        start_gather(0, 0)

        def step(s, carry):
            b = lax.rem(s, 2)

            @pl.when(b == 0)
            def _():
                wait_gather(s, 0)

                @pl.when(s + 1 < seq)
                def _():
                    start_gather(s + 1, 1)

                @pl.when(s >= 2)
                def _():
                    wait_store(s - 2, 0)
                transpose(0)
                start_store(s, 0)

            @pl.when(b == 1)
            def _():
                wait_gather(s, 1)

                @pl.when(s + 1 < seq)
                def _():
                    start_gather(s + 1, 0)

                @pl.when(s >= 2)
                def _():
                    wait_store(s - 2, 1)
                transpose(1)
                start_store(s, 1)

            return carry

        lax.fori_loop(0, seq, step, 0)
        wait_store(seq - 2, (seq - 2) % 2)
        wait_store(seq - 1, (seq - 1) % 2)

    out_phys = gather_k(ids, rdup)
    return jnp.transpose(out_phys, (0, 2, 1))


# speedup vs baseline: 3.4888x; 1.0519x over previous
import functools

import jax
import jax.numpy as jnp
from jax import lax
from jax.experimental import pallas as pl
from jax.experimental.pallas import tpu as pltpu
from jax.experimental.pallas import tpu_sc as plsc

NUM_CORES = 2
NUM_SUBCORES = 16
NUM_WORKERS = NUM_CORES * NUM_SUBCORES
LANES = 16


def kernel(input_ids, table):
    seq, batch = input_ids.shape
    vocab, dim = table.shape
    assert batch == NUM_WORKERS * 128
    assert 2 * dim == 128

    ids = input_ids.astype(jnp.int32)
    rdup = jnp.pad(table, ((0, 0), (0, 128 - dim)))

    mesh = plsc.VectorSubcoreMesh(
        core_axis_name="c", subcore_axis_name="s",
        num_cores=NUM_CORES, num_subcores=NUM_SUBCORES)

    @functools.partial(
        pl.kernel,
        mesh=mesh,
        out_type=jax.ShapeDtypeStruct((seq, dim, batch), jnp.float32),
        scratch_types=[
            pltpu.VMEM((seq, 128), jnp.int32),
            pltpu.VMEM((128, 128), jnp.float32),
            pltpu.VMEM((128, 128), jnp.float32),
            pltpu.VMEM((dim, 128), jnp.float32),
            pltpu.VMEM((dim, 128), jnp.float32),
            pltpu.SemaphoreType.DMA,
            pltpu.SemaphoreType.DMA,
            pltpu.SemaphoreType.DMA,
            pltpu.SemaphoreType.DMA,
        ],
        compiler_params=pltpu.CompilerParams(needs_layout_passes=False),
    )
    def gather_k(ids_hbm, r_hbm, out_hbm, idxv, gb0, gb1, ob0, ob1,
                 gs0, gs1, ss0, ss1):
        gbufs, obufs = (gb0, gb1), (ob0, ob1)
        gsems, ssems = (gs0, gs1), (ss0, ss1)
        wid = lax.axis_index("s") * NUM_CORES + lax.axis_index("c")
        col = wid * 128
        rows16 = [lax.iota(jnp.int32, LANES) + LANES * g for g in range(8)]
        pltpu.sync_copy(ids_hbm.at[:, pl.ds(col, 128)], idxv)

        def start_gather(s, b):
            pltpu.async_copy(r_hbm.at[idxv.at[s]], gbufs[b], gsems[b])

        def wait_gather(s, b):
            pltpu.make_async_copy(
                r_hbm.at[idxv.at[s]], gbufs[b], gsems[b]).wait()

        def dst(s):
            return out_hbm.at[s].at[:, pl.ds(col, 128)]

        def start_store(s, b):
            pltpu.async_copy(obufs[b], dst(s), ssems[b])

        def wait_store(s, b):
            pltpu.make_async_copy(obufs[b], dst(s), ssems[b]).wait()

        iota16 = lax.iota(jnp.int32, LANES)
        perm16 = [(lax.iota(jnp.int32, LANES) + d) % LANES for d in range(LANES)]

        def transpose(b):
            gbuf, obuf = gbufs[b], obufs[b]

            @plsc.parallel_loop(0, 8, unroll=2)
            def jblk(jb):
                rows_a = jnp.zeros((LANES,), jnp.int32) + jb * LANES + iota16
                for cq in range(dim // LANES):
                    for d in range(LANES):
                        cols_a = perm16[d] + cq * LANES
                        vec = plsc.load_gather(gbuf, [rows_a, cols_a])
                        plsc.store_scatter(obuf, [cols_a, rows_a], vec)

        start_gather(0, 0)

        def step(s, carry):
            b = lax.rem(s, 2)

            @pl.when(b == 0)
            def _():
                wait_gather(s, 0)

                @pl.when(s + 1 < seq)
                def _():
                    start_gather(s + 1, 1)

                @pl.when(s >= 2)
                def _():
                    wait_store(s - 2, 0)
                transpose(0)
                start_store(s, 0)

            @pl.when(b == 1)
            def _():
                wait_gather(s, 1)

                @pl.when(s + 1 < seq)
                def _():
                    start_gather(s + 1, 0)

                @pl.when(s >= 2)
                def _():
                    wait_store(s - 2, 1)
                transpose(1)
                start_store(s, 1)

            return carry

        lax.fori_loop(0, seq, step, 0)
        wait_store(seq - 2, (seq - 2) % 2)
        wait_store(seq - 1, (seq - 1) % 2)

    out_phys = gather_k(ids, rdup)
    return jnp.transpose(out_phys, (0, 2, 1))
